# Initial kernel scaffold; baseline (speedup 1.0000x reference)
import jax
import jax.numpy as jnp
from jax.experimental import pallas as pl


def kernel(cycle_curve_data, logits, moe_masks, W, b):
    raise NotImplementedError("write your pallas kernel here")


# speedup vs baseline: 1.4132x; 1.4132x over previous
import jax
import jax.numpy as jnp
from jax.experimental import pallas as pl
from jax.experimental.pallas import tpu as pltpu

B, L, CLEN, E, TOP_K, D_MODEL = 256, 8, 128, 64, 2, 128
DIN = 3 * CLEN
R = B * L
EPS = 1e-9


def _moe_kernel(logits_ref, masks_ref, flat_ref, W_ref, b_ref, out_ref):
    logits = logits_ref[...]
    maskf = (masks_ref[...] == 1).astype(jnp.float32)
    rowmax = jnp.max(logits, axis=1, keepdims=True)
    ex = jnp.exp(logits - rowmax)
    g = ex / jnp.sum(ex, axis=1, keepdims=True) * maskf

    iota = jax.lax.broadcasted_iota(jnp.int32, (B, E), 1)
    v1 = jnp.max(g, axis=1, keepdims=True)
    idx1 = jnp.min(jnp.where(g == v1, iota, E), axis=1, keepdims=True)
    oh1 = iota == idx1
    g2 = jnp.where(oh1, -1.0, g)
    v2 = jnp.max(g2, axis=1, keepdims=True)
    idx2 = jnp.min(jnp.where(g2 == v2, iota, E), axis=1, keepdims=True)
    oh2 = iota == idx2
    denom = v1 + v2 + EPS
    c = (jnp.where(oh1, v1, 0.0) + jnp.where(oh2, v2, 0.0)) / denom

    c2 = jnp.broadcast_to(c[:, None, :], (B, L, E)).reshape(R, E)

    acc = jnp.dot(c2, b_ref[...], preferred_element_type=jnp.float32)
    flat_bf = flat_ref[...].astype(jnp.bfloat16)
    for e in range(E):
        y = jnp.dot(flat_bf, W_ref[e].astype(jnp.bfloat16),
                    preferred_element_type=jnp.float32)
        acc = acc + c2[:, e:e + 1] * y
    out_ref[...] = acc.astype(jnp.bfloat16)


def kernel(cycle_curve_data, logits, moe_masks, W, b):
    flat2 = cycle_curve_data.reshape(R, DIN)
    out = pl.pallas_call(
        _moe_kernel,
        out_shape=jax.ShapeDtypeStruct((R, D_MODEL), jnp.bfloat16),
    )(logits, moe_masks, flat2, W, b)
    return out.reshape(B, L, D_MODEL)


# speedup vs baseline: 1.7518x; 1.2396x over previous
import jax
import jax.numpy as jnp
from jax.experimental import pallas as pl
from jax.experimental.pallas import tpu as pltpu

B, L, CLEN, E, TOP_K, D_MODEL = 256, 8, 128, 64, 2, 128
DIN = 3 * CLEN
R = B * L
EPS = 1e-9

EB = 8
NSTEPS = E // EB


def _routing(logits, masks):
    maskf = (masks == 1).astype(jnp.float32)
    rowmax = jnp.max(logits, axis=1, keepdims=True)
    ex = jnp.exp(logits - rowmax)
    g = ex / jnp.sum(ex, axis=1, keepdims=True) * maskf

    iota = jax.lax.broadcasted_iota(jnp.int32, (B, E), 1)
    v1 = jnp.max(g, axis=1, keepdims=True)
    idx1 = jnp.min(jnp.where(g == v1, iota, E), axis=1, keepdims=True)
    oh1 = iota == idx1
    g2 = jnp.where(oh1, -1.0, g)
    v2 = jnp.max(g2, axis=1, keepdims=True)
    idx2 = jnp.min(jnp.where(g2 == v2, iota, E), axis=1, keepdims=True)
    oh2 = iota == idx2
    denom = v1 + v2 + EPS
    return (jnp.where(oh1, v1, 0.0) + jnp.where(oh2, v2, 0.0)) / denom


def _moe_kernel(logits_ref, masks_ref, flat_ref, W_ref, b_ref, out_ref,
                acc_ref, c2_ref, flat_bf_ref):
    step = pl.program_id(0)

    @pl.when(step == 0)
    def _prologue():
        c = _routing(logits_ref[...], masks_ref[...])
        c2 = jnp.broadcast_to(c[:, None, :], (B, L, E)).reshape(R, E)
        for s in range(NSTEPS):
            c2_ref[s] = c2[:, s * EB:(s + 1) * EB]
        flat_bf_ref[...] = flat_ref[...].astype(jnp.bfloat16)
        acc_ref[...] = jnp.dot(c2, b_ref[...],
                               preferred_element_type=jnp.float32)

    flat_bf = flat_bf_ref[...]
    cs = c2_ref[step]
    acc = acc_ref[...]
    for j in range(EB // 2):
        wp = jnp.concatenate(
            [W_ref[2 * j], W_ref[2 * j + 1]], axis=1).astype(jnp.bfloat16)
        y = jnp.dot(flat_bf, wp, preferred_element_type=jnp.float32)
        acc = (acc
               + cs[:, 2 * j:2 * j + 1] * y[:, :D_MODEL]
               + cs[:, 2 * j + 1:2 * j + 2] * y[:, D_MODEL:])
    acc_ref[...] = acc

    @pl.when(step == NSTEPS - 1)
    def _epilogue():
        out_ref[...] = acc.astype(jnp.bfloat16)


def kernel(cycle_curve_data, logits, moe_masks, W, b):
    flat2 = cycle_curve_data.reshape(R, DIN)
    out = pl.pallas_call(
        _moe_kernel,
        grid=(NSTEPS,),
        in_specs=[
            pl.BlockSpec((B, E), lambda i: (0, 0)),
            pl.BlockSpec((B, E), lambda i: (0, 0)),
            pl.BlockSpec((R, DIN), lambda i: (0, 0)),
            pl.BlockSpec((EB, DIN, D_MODEL), lambda i: (i, 0, 0)),
            pl.BlockSpec((E, D_MODEL), lambda i: (0, 0)),
        ],
        out_specs=pl.BlockSpec((R, D_MODEL), lambda i: (0, 0)),
        out_shape=jax.ShapeDtypeStruct((R, D_MODEL), jnp.bfloat16),
        scratch_shapes=[
            pltpu.VMEM((R, D_MODEL), jnp.float32),
            pltpu.VMEM((NSTEPS, R, EB), jnp.float32),
            pltpu.VMEM((R, DIN), jnp.bfloat16),
        ],
    )(logits, moe_masks, flat2, W, b)
    return out.reshape(B, L, D_MODEL)


# speedup vs baseline: 1.8344x; 1.0471x over previous
import jax
import jax.numpy as jnp
from jax.experimental import pallas as pl
from jax.experimental.pallas import tpu as pltpu

B, L, CLEN, E, TOP_K, D_MODEL = 256, 8, 128, 64, 2, 128
DIN = 3 * CLEN
R = B * L
EPS = 1e-9

EB = 8
NSTEPS = E // EB

_DN_T = (((0,), (0,)), ((), ()))


def _routing(logits, masks):
    maskf = (masks == 1).astype(jnp.float32)
    rowmax = jnp.max(logits, axis=1, keepdims=True)
    ex = jnp.exp(logits - rowmax)
    g = ex / jnp.sum(ex, axis=1, keepdims=True) * maskf

    iota = jax.lax.broadcasted_iota(jnp.int32, (B, E), 1)
    v1 = jnp.max(g, axis=1, keepdims=True)
    idx1 = jnp.min(jnp.where(g == v1, iota, E), axis=1, keepdims=True)
    oh1 = iota == idx1
    g2 = jnp.where(oh1, -1.0, g)
    v2 = jnp.max(g2, axis=1, keepdims=True)
    idx2 = jnp.min(jnp.where(g2 == v2, iota, E), axis=1, keepdims=True)
    oh2 = iota == idx2
    denom = v1 + v2 + EPS
    return (jnp.where(oh1, v1, 0.0) + jnp.where(oh2, v2, 0.0)) / denom


def _moe_kernel(logits_ref, masks_ref, flat_ref, W_ref, b_ref, out_ref,
                accT_ref, cs_ref, xT_ref):
    step = pl.program_id(0)

    @pl.when(step == 0)
    def _prologue():
        c = _routing(logits_ref[...], masks_ref[...])
        lane_b = jax.lax.broadcasted_iota(jnp.int32, (B, R), 1) // L
        sub_b = jax.lax.broadcasted_iota(jnp.int32, (B, R), 0)
        exp_mat = (lane_b == sub_b).astype(jnp.bfloat16)
        c2T = jax.lax.dot_general(
            c.astype(jnp.bfloat16), exp_mat, _DN_T,
            preferred_element_type=jnp.float32)
        for s in range(NSTEPS):
            cs_ref[s] = c2T[s * EB:(s + 1) * EB]
        xT_ref[...] = flat_ref[...].T.astype(jnp.bfloat16)
        accT_ref[...] = jax.lax.dot_general(
            b_ref[...], c2T, _DN_T, preferred_element_type=jnp.float32)

    xT = xT_ref[...]
    cs = cs_ref[step]
    acc = accT_ref[...]
    for j in range(EB):
        w = W_ref[j].astype(jnp.bfloat16)
        y = jax.lax.dot_general(w, xT, _DN_T,
                                preferred_element_type=jnp.float32)
        acc = acc + cs[j:j + 1, :] * y
    accT_ref[...] = acc

    @pl.when(step == NSTEPS - 1)
    def _epilogue():
        out_ref[...] = acc.astype(jnp.bfloat16).T


def kernel(cycle_curve_data, logits, moe_masks, W, b):
    flat2 = cycle_curve_data.reshape(R, DIN)
    out = pl.pallas_call(
        _moe_kernel,
        grid=(NSTEPS,),
        in_specs=[
            pl.BlockSpec((B, E), lambda i: (0, 0)),
            pl.BlockSpec((B, E), lambda i: (0, 0)),
            pl.BlockSpec((R, DIN), lambda i: (0, 0)),
            pl.BlockSpec((EB, DIN, D_MODEL), lambda i: (i, 0, 0)),
            pl.BlockSpec((E, D_MODEL), lambda i: (0, 0)),
        ],
        out_specs=pl.BlockSpec((R, D_MODEL), lambda i: (0, 0)),
        out_shape=jax.ShapeDtypeStruct((R, D_MODEL), jnp.bfloat16),
        scratch_shapes=[
            pltpu.VMEM((D_MODEL, R), jnp.float32),
            pltpu.VMEM((NSTEPS, EB, R), jnp.float32),
            pltpu.VMEM((DIN, R), jnp.bfloat16),
        ],
    )(logits, moe_masks, flat2, W, b)
    return out.reshape(B, L, D_MODEL)
